# baseline (speedup 1.0000x reference)
import functools

import jax
import jax.numpy as jnp
from jax import lax
from jax.experimental import pallas as pl
from jax.experimental.pallas import tpu as pltpu
from jax.experimental.pallas import tpu_sc as plsc

B = 16


def _sc_lookup(camindex, idindex, dataset_type,
               wcam1f, bcam1f, wident1f, bident1f,
               wcam2f, bcam2f, wident2f, bident2f):
    mesh = plsc.VectorSubcoreMesh(core_axis_name="c", subcore_axis_name="s")
    table_sizes = [wcam1f.size, bcam1f.size, wident1f.size, bident1f.size,
                   wcam2f.size, bcam2f.size, wident2f.size, bident2f.size]

    @functools.partial(
        pl.kernel,
        mesh=mesh,
        compiler_params=pltpu.CompilerParams(needs_layout_passes=False),
        out_type=[jax.ShapeDtypeStruct((3, B), jnp.float32),
                  jax.ShapeDtypeStruct((3, B), jnp.float32)],
        scratch_types=[
            pltpu.VMEM((B,), jnp.int32),
            pltpu.VMEM((B,), jnp.int32),
            pltpu.VMEM((B,), jnp.int32),
        ] + [pltpu.VMEM((n,), jnp.float32) for n in table_sizes] + [
            pltpu.VMEM((3, B), jnp.float32),
            pltpu.VMEM((3, B), jnp.float32),
        ],
    )
    def lookup(cam_h, id_h, dt_h,
               wc1_h, bc1_h, wi1_h, bi1_h, wc2_h, bc2_h, wi2_h, bi2_h,
               w_out, b_out,
               cam_v, id_v, dt_v,
               wc1_v, bc1_v, wi1_v, bi1_v, wc2_v, bc2_v, wi2_v, bi2_v,
               w_v, b_v):
        wid = lax.axis_index("s") * 2 + lax.axis_index("c")

        @pl.when(wid == 0)
        def _():
            pltpu.sync_copy(cam_h, cam_v)
            pltpu.sync_copy(id_h, id_v)
            pltpu.sync_copy(dt_h, dt_v)
            pltpu.sync_copy(wc1_h, wc1_v)
            pltpu.sync_copy(bc1_h, bc1_v)
            pltpu.sync_copy(wi1_h, wi1_v)
            pltpu.sync_copy(bi1_h, bi1_v)
            pltpu.sync_copy(wc2_h, wc2_v)
            pltpu.sync_copy(bc2_h, bc2_v)
            pltpu.sync_copy(wi2_h, wi2_v)
            pltpu.sync_copy(bi2_h, bi2_v)

            cam3 = cam_v[...] * 3
            id3 = id_v[...] * 3
            use1 = dt_v[...] == 0
            for c in range(3):
                w1 = (plsc.load_gather(wc1_v, [cam3 + c]) +
                      plsc.load_gather(wi1_v, [id3 + c]))
                w2 = (plsc.load_gather(wc2_v, [cam3 + c]) +
                      plsc.load_gather(wi2_v, [id3 + c]))
                b1 = (plsc.load_gather(bc1_v, [cam3 + c]) +
                      plsc.load_gather(bi1_v, [id3 + c]))
                b2 = (plsc.load_gather(bc2_v, [cam3 + c]) +
                      plsc.load_gather(bi2_v, [id3 + c]))
                w_v[c, :] = jnp.where(use1, w1, w2)
                b_v[c, :] = jnp.where(use1, b1, b2)
            pltpu.sync_copy(w_v, w_out)
            pltpu.sync_copy(b_v, b_out)

    return lookup(camindex, idindex, dataset_type,
                  wcam1f, bcam1f, wident1f, bident1f,
                  wcam2f, bcam2f, wident2f, bident2f)


def _affine_body(w_ref, b_ref, img_ref, out_ref):
    b_i = pl.program_id(0)
    c_i = pl.program_id(1)
    out_ref[...] = img_ref[...] * w_ref[c_i, b_i] + b_ref[c_i, b_i]


def _tc_affine(w, b, image):
    return pl.pallas_call(
        _affine_body,
        grid=(B, 3),
        in_specs=[
            pl.BlockSpec(memory_space=pltpu.SMEM),
            pl.BlockSpec(memory_space=pltpu.SMEM),
            pl.BlockSpec((1, 1, 512, 512), lambda bi, ci: (bi, ci, 0, 0)),
        ],
        out_specs=pl.BlockSpec((1, 1, 512, 512), lambda bi, ci: (bi, ci, 0, 0)),
        out_shape=jax.ShapeDtypeStruct(image.shape, image.dtype),
        compiler_params=pltpu.CompilerParams(
            dimension_semantics=("parallel", "parallel")),
    )(w, b, image)


@jax.jit
def kernel(image, camindex, idindex, dataset_type,
           wcam1, bcam1, wident1, bident1,
           wcam2, bcam2, wident2, bident2):
    w, b = _sc_lookup(camindex, idindex, dataset_type,
                      wcam1.reshape(-1), bcam1.reshape(-1),
                      wident1.reshape(-1), bident1.reshape(-1),
                      wcam2.reshape(-1), bcam2.reshape(-1),
                      wident2.reshape(-1), bident2.reshape(-1))
    return _tc_affine(w, b, image)


# speedup vs baseline: 1.2015x; 1.2015x over previous
import functools

import jax
import jax.numpy as jnp
from jax import lax
from jax.experimental import pallas as pl
from jax.experimental.pallas import tpu as pltpu
from jax.experimental.pallas import tpu_sc as plsc

B = 16


def _sc_lookup(camindex, idindex, dataset_type,
               wcam1f, bcam1f, wident1f, bident1f,
               wcam2f, bcam2f, wident2f, bident2f):
    mesh = plsc.VectorSubcoreMesh(core_axis_name="c", subcore_axis_name="s")
    table_sizes = [wcam1f.size, bcam1f.size, wident1f.size, bident1f.size,
                   wcam2f.size, bcam2f.size, wident2f.size, bident2f.size]

    @functools.partial(
        pl.kernel,
        mesh=mesh,
        compiler_params=pltpu.CompilerParams(needs_layout_passes=False),
        out_type=[jax.ShapeDtypeStruct((3, B), jnp.float32),
                  jax.ShapeDtypeStruct((3, B), jnp.float32)],
        scratch_types=[
            pltpu.VMEM((B,), jnp.int32),
            pltpu.VMEM((B,), jnp.int32),
            pltpu.VMEM((B,), jnp.int32),
        ] + [pltpu.VMEM((n,), jnp.float32) for n in table_sizes] + [
            pltpu.VMEM((3, B), jnp.float32),
            pltpu.VMEM((3, B), jnp.float32),
        ],
    )
    def lookup(cam_h, id_h, dt_h,
               wc1_h, bc1_h, wi1_h, bi1_h, wc2_h, bc2_h, wi2_h, bi2_h,
               w_out, b_out,
               cam_v, id_v, dt_v,
               wc1_v, bc1_v, wi1_v, bi1_v, wc2_v, bc2_v, wi2_v, bi2_v,
               w_v, b_v):
        wid = lax.axis_index("s") * 2 + lax.axis_index("c")

        @pl.when(wid == 0)
        def _():
            pltpu.sync_copy(cam_h, cam_v)
            pltpu.sync_copy(id_h, id_v)
            pltpu.sync_copy(dt_h, dt_v)
            pltpu.sync_copy(wc1_h, wc1_v)
            pltpu.sync_copy(bc1_h, bc1_v)
            pltpu.sync_copy(wi1_h, wi1_v)
            pltpu.sync_copy(bi1_h, bi1_v)
            pltpu.sync_copy(wc2_h, wc2_v)
            pltpu.sync_copy(bc2_h, bc2_v)
            pltpu.sync_copy(wi2_h, wi2_v)
            pltpu.sync_copy(bi2_h, bi2_v)

            cam3 = cam_v[...] * 3
            id3 = id_v[...] * 3
            use1 = dt_v[...] == 0
            for c in range(3):
                w1 = (plsc.load_gather(wc1_v, [cam3 + c]) +
                      plsc.load_gather(wi1_v, [id3 + c]))
                w2 = (plsc.load_gather(wc2_v, [cam3 + c]) +
                      plsc.load_gather(wi2_v, [id3 + c]))
                b1 = (plsc.load_gather(bc1_v, [cam3 + c]) +
                      plsc.load_gather(bi1_v, [id3 + c]))
                b2 = (plsc.load_gather(bc2_v, [cam3 + c]) +
                      plsc.load_gather(bi2_v, [id3 + c]))
                w_v[c, :] = jnp.where(use1, w1, w2)
                b_v[c, :] = jnp.where(use1, b1, b2)
            pltpu.sync_copy(w_v, w_out)
            pltpu.sync_copy(b_v, b_out)

    return lookup(camindex, idindex, dataset_type,
                  wcam1f, bcam1f, wident1f, bident1f,
                  wcam2f, bcam2f, wident2f, bident2f)


def _affine_body(w_ref, b_ref, img_ref, out_ref):
    b_i = pl.program_id(0)
    for c in range(3):
        out_ref[0, c] = img_ref[0, c] * w_ref[c, b_i] + b_ref[c, b_i]


def _tc_affine(w, b, image):
    return pl.pallas_call(
        _affine_body,
        grid=(B,),
        in_specs=[
            pl.BlockSpec(memory_space=pltpu.SMEM),
            pl.BlockSpec(memory_space=pltpu.SMEM),
            pl.BlockSpec((1, 3, 512, 512), lambda bi: (bi, 0, 0, 0)),
        ],
        out_specs=pl.BlockSpec((1, 3, 512, 512), lambda bi: (bi, 0, 0, 0)),
        out_shape=jax.ShapeDtypeStruct(image.shape, image.dtype),
        compiler_params=pltpu.CompilerParams(
            dimension_semantics=("parallel",)),
    )(w, b, image)


@jax.jit
def kernel(image, camindex, idindex, dataset_type,
           wcam1, bcam1, wident1, bident1,
           wcam2, bcam2, wident2, bident2):
    w, b = _sc_lookup(camindex, idindex, dataset_type,
                      wcam1.reshape(-1), bcam1.reshape(-1),
                      wident1.reshape(-1), bident1.reshape(-1),
                      wcam2.reshape(-1), bcam2.reshape(-1),
                      wident2.reshape(-1), bident2.reshape(-1))
    return _tc_affine(w, b, image)


# speedup vs baseline: 2.0946x; 1.7433x over previous
import functools

import jax
import jax.numpy as jnp
from jax import lax
from jax.experimental import pallas as pl
from jax.experimental.pallas import tpu as pltpu
from jax.experimental.pallas import tpu_sc as plsc

B = 16


def _sc_lookup(camindex, idindex, dataset_type,
               wcam1f, bcam1f, wident1f, bident1f,
               wcam2f, bcam2f, wident2f, bident2f):
    mesh = plsc.VectorSubcoreMesh(core_axis_name="c", subcore_axis_name="s")
    table_sizes = [wcam1f.size, bcam1f.size, wident1f.size, bident1f.size,
                   wcam2f.size, bcam2f.size, wident2f.size, bident2f.size]

    @functools.partial(
        pl.kernel,
        mesh=mesh,
        compiler_params=pltpu.CompilerParams(needs_layout_passes=False),
        out_type=[jax.ShapeDtypeStruct((3, B), jnp.float32),
                  jax.ShapeDtypeStruct((3, B), jnp.float32)],
        scratch_types=[
            pltpu.VMEM((B,), jnp.int32),
            pltpu.VMEM((B,), jnp.int32),
            pltpu.VMEM((B,), jnp.int32),
        ] + [pltpu.VMEM((n,), jnp.float32) for n in table_sizes] + [
            pltpu.VMEM((3, B), jnp.float32),
            pltpu.VMEM((3, B), jnp.float32),
        ],
    )
    def lookup(cam_h, id_h, dt_h,
               wc1_h, bc1_h, wi1_h, bi1_h, wc2_h, bc2_h, wi2_h, bi2_h,
               w_out, b_out,
               cam_v, id_v, dt_v,
               wc1_v, bc1_v, wi1_v, bi1_v, wc2_v, bc2_v, wi2_v, bi2_v,
               w_v, b_v):
        wid = lax.axis_index("s") * 2 + lax.axis_index("c")

        @pl.when(wid == 0)
        def _():
            pltpu.sync_copy(cam_h, cam_v)
            pltpu.sync_copy(id_h, id_v)
            pltpu.sync_copy(dt_h, dt_v)
            pltpu.sync_copy(wc1_h, wc1_v)
            pltpu.sync_copy(bc1_h, bc1_v)
            pltpu.sync_copy(wi1_h, wi1_v)
            pltpu.sync_copy(bi1_h, bi1_v)
            pltpu.sync_copy(wc2_h, wc2_v)
            pltpu.sync_copy(bc2_h, bc2_v)
            pltpu.sync_copy(wi2_h, wi2_v)
            pltpu.sync_copy(bi2_h, bi2_v)

            cam3 = cam_v[...] * 3
            id3 = id_v[...] * 3
            use1 = dt_v[...] == 0
            for c in range(3):
                w1 = (plsc.load_gather(wc1_v, [cam3 + c]) +
                      plsc.load_gather(wi1_v, [id3 + c]))
                w2 = (plsc.load_gather(wc2_v, [cam3 + c]) +
                      plsc.load_gather(wi2_v, [id3 + c]))
                b1 = (plsc.load_gather(bc1_v, [cam3 + c]) +
                      plsc.load_gather(bi1_v, [id3 + c]))
                b2 = (plsc.load_gather(bc2_v, [cam3 + c]) +
                      plsc.load_gather(bi2_v, [id3 + c]))
                w_v[c, :] = jnp.where(use1, w1, w2)
                b_v[c, :] = jnp.where(use1, b1, b2)
            pltpu.sync_copy(w_v, w_out)
            pltpu.sync_copy(b_v, b_out)

    return lookup(camindex, idindex, dataset_type,
                  wcam1f, bcam1f, wident1f, bident1f,
                  wcam2f, bcam2f, wident2f, bident2f)


def _affine_body(w_ref, b_ref, img_ref, out_ref):
    b_i = pl.program_id(0)
    for c in range(3):
        out_ref[0, c] = img_ref[0, c] * w_ref[c, b_i] + b_ref[c, b_i]


def _tc_affine(w, b, image):
    return pl.pallas_call(
        _affine_body,
        grid=(B,),
        in_specs=[
            pl.BlockSpec(memory_space=pltpu.SMEM),
            pl.BlockSpec(memory_space=pltpu.SMEM),
            pl.BlockSpec((1, 3, 512, 512), lambda bi: (bi, 0, 0, 0)),
        ],
        out_specs=pl.BlockSpec((1, 3, 512, 512), lambda bi: (bi, 0, 0, 0)),
        out_shape=jax.ShapeDtypeStruct(image.shape, image.dtype),
        compiler_params=pltpu.CompilerParams(
            dimension_semantics=("parallel",)),
    )(w, b, image)


@jax.jit
def kernel(image, camindex, idindex, dataset_type,
           wcam1, bcam1, wident1, bident1,
           wcam2, bcam2, wident2, bident2):
    use_sc = False
    if use_sc:
        w, b = _sc_lookup(camindex, idindex, dataset_type,
                          wcam1.reshape(-1), bcam1.reshape(-1),
                          wident1.reshape(-1), bident1.reshape(-1),
                          wcam2.reshape(-1), bcam2.reshape(-1),
                          wident2.reshape(-1), bident2.reshape(-1))
    else:
        w1 = jnp.take(wcam1, camindex, axis=0) + jnp.take(wident1, idindex, axis=0)
        b1 = jnp.take(bcam1, camindex, axis=0) + jnp.take(bident1, idindex, axis=0)
        w2 = jnp.take(wcam2, camindex, axis=0) + jnp.take(wident2, idindex, axis=0)
        b2 = jnp.take(bcam2, camindex, axis=0) + jnp.take(bident2, idindex, axis=0)
        mask = (dataset_type == 0)[:, None]
        w = jnp.where(mask, w1, w2).T
        b = jnp.where(mask, b1, b2).T
    return _tc_affine(w, b, image)


# speedup vs baseline: 2.1589x; 1.0307x over previous
import functools

import jax
import jax.numpy as jnp
from jax import lax
from jax.experimental import pallas as pl
from jax.experimental.pallas import tpu as pltpu
from jax.experimental.pallas import tpu_sc as plsc

B = 16


def _sc_lookup(camindex, idindex, dataset_type,
               wcam1f, bcam1f, wident1f, bident1f,
               wcam2f, bcam2f, wident2f, bident2f):
    mesh = plsc.VectorSubcoreMesh(core_axis_name="c", subcore_axis_name="s")
    table_sizes = [wcam1f.size, bcam1f.size, wident1f.size, bident1f.size,
                   wcam2f.size, bcam2f.size, wident2f.size, bident2f.size]

    @functools.partial(
        pl.kernel,
        mesh=mesh,
        compiler_params=pltpu.CompilerParams(needs_layout_passes=False),
        out_type=[jax.ShapeDtypeStruct((3, B), jnp.float32),
                  jax.ShapeDtypeStruct((3, B), jnp.float32)],
        scratch_types=[
            pltpu.VMEM((B,), jnp.int32),
            pltpu.VMEM((B,), jnp.int32),
            pltpu.VMEM((B,), jnp.int32),
        ] + [pltpu.VMEM((n,), jnp.float32) for n in table_sizes] + [
            pltpu.VMEM((3, B), jnp.float32),
            pltpu.VMEM((3, B), jnp.float32),
        ],
    )
    def lookup(cam_h, id_h, dt_h,
               wc1_h, bc1_h, wi1_h, bi1_h, wc2_h, bc2_h, wi2_h, bi2_h,
               w_out, b_out,
               cam_v, id_v, dt_v,
               wc1_v, bc1_v, wi1_v, bi1_v, wc2_v, bc2_v, wi2_v, bi2_v,
               w_v, b_v):
        wid = lax.axis_index("s") * 2 + lax.axis_index("c")

        @pl.when(wid == 0)
        def _():
            pltpu.sync_copy(cam_h, cam_v)
            pltpu.sync_copy(id_h, id_v)
            pltpu.sync_copy(dt_h, dt_v)
            pltpu.sync_copy(wc1_h, wc1_v)
            pltpu.sync_copy(bc1_h, bc1_v)
            pltpu.sync_copy(wi1_h, wi1_v)
            pltpu.sync_copy(bi1_h, bi1_v)
            pltpu.sync_copy(wc2_h, wc2_v)
            pltpu.sync_copy(bc2_h, bc2_v)
            pltpu.sync_copy(wi2_h, wi2_v)
            pltpu.sync_copy(bi2_h, bi2_v)

            cam3 = cam_v[...] * 3
            id3 = id_v[...] * 3
            use1 = dt_v[...] == 0
            for c in range(3):
                w1 = (plsc.load_gather(wc1_v, [cam3 + c]) +
                      plsc.load_gather(wi1_v, [id3 + c]))
                w2 = (plsc.load_gather(wc2_v, [cam3 + c]) +
                      plsc.load_gather(wi2_v, [id3 + c]))
                b1 = (plsc.load_gather(bc1_v, [cam3 + c]) +
                      plsc.load_gather(bi1_v, [id3 + c]))
                b2 = (plsc.load_gather(bc2_v, [cam3 + c]) +
                      plsc.load_gather(bi2_v, [id3 + c]))
                w_v[c, :] = jnp.where(use1, w1, w2)
                b_v[c, :] = jnp.where(use1, b1, b2)
            pltpu.sync_copy(w_v, w_out)
            pltpu.sync_copy(b_v, b_out)

    return lookup(camindex, idindex, dataset_type,
                  wcam1f, bcam1f, wident1f, bident1f,
                  wcam2f, bcam2f, wident2f, bident2f)


NB = 2


def _affine_body(w_ref, b_ref, img_ref, out_ref):
    b_i = pl.program_id(0)
    for j in range(NB):
        for c in range(3):
            out_ref[j, c] = (img_ref[j, c] * w_ref[c, b_i * NB + j]
                             + b_ref[c, b_i * NB + j])


def _tc_affine(w, b, image):
    return pl.pallas_call(
        _affine_body,
        grid=(B // NB,),
        in_specs=[
            pl.BlockSpec(memory_space=pltpu.SMEM),
            pl.BlockSpec(memory_space=pltpu.SMEM),
            pl.BlockSpec((NB, 3, 512, 512), lambda bi: (bi, 0, 0, 0)),
        ],
        out_specs=pl.BlockSpec((NB, 3, 512, 512), lambda bi: (bi, 0, 0, 0)),
        out_shape=jax.ShapeDtypeStruct(image.shape, image.dtype),
        compiler_params=pltpu.CompilerParams(
            dimension_semantics=("parallel",)),
    )(w, b, image)


@jax.jit
def kernel(image, camindex, idindex, dataset_type,
           wcam1, bcam1, wident1, bident1,
           wcam2, bcam2, wident2, bident2):
    use_sc = False
    if use_sc:
        w, b = _sc_lookup(camindex, idindex, dataset_type,
                          wcam1.reshape(-1), bcam1.reshape(-1),
                          wident1.reshape(-1), bident1.reshape(-1),
                          wcam2.reshape(-1), bcam2.reshape(-1),
                          wident2.reshape(-1), bident2.reshape(-1))
    else:
        w1 = jnp.take(wcam1, camindex, axis=0) + jnp.take(wident1, idindex, axis=0)
        b1 = jnp.take(bcam1, camindex, axis=0) + jnp.take(bident1, idindex, axis=0)
        w2 = jnp.take(wcam2, camindex, axis=0) + jnp.take(wident2, idindex, axis=0)
        b2 = jnp.take(bcam2, camindex, axis=0) + jnp.take(bident2, idindex, axis=0)
        mask = (dataset_type == 0)[:, None]
        w = jnp.where(mask, w1, w2).T
        b = jnp.where(mask, b1, b2).T
    return _tc_affine(w, b, image)


# speedup vs baseline: 2.2210x; 1.0288x over previous
import functools

import jax
import jax.numpy as jnp
from jax import lax
from jax.experimental import pallas as pl
from jax.experimental.pallas import tpu as pltpu
from jax.experimental.pallas import tpu_sc as plsc

B = 16


def _sc_lookup(camindex, idindex, dataset_type,
               wcam1f, bcam1f, wident1f, bident1f,
               wcam2f, bcam2f, wident2f, bident2f):
    mesh = plsc.VectorSubcoreMesh(core_axis_name="c", subcore_axis_name="s")
    table_sizes = [wcam1f.size, bcam1f.size, wident1f.size, bident1f.size,
                   wcam2f.size, bcam2f.size, wident2f.size, bident2f.size]

    @functools.partial(
        pl.kernel,
        mesh=mesh,
        compiler_params=pltpu.CompilerParams(needs_layout_passes=False),
        out_type=[jax.ShapeDtypeStruct((3, B), jnp.float32),
                  jax.ShapeDtypeStruct((3, B), jnp.float32)],
        scratch_types=[
            pltpu.VMEM((B,), jnp.int32),
            pltpu.VMEM((B,), jnp.int32),
            pltpu.VMEM((B,), jnp.int32),
        ] + [pltpu.VMEM((n,), jnp.float32) for n in table_sizes] + [
            pltpu.VMEM((3, B), jnp.float32),
            pltpu.VMEM((3, B), jnp.float32),
        ],
    )
    def lookup(cam_h, id_h, dt_h,
               wc1_h, bc1_h, wi1_h, bi1_h, wc2_h, bc2_h, wi2_h, bi2_h,
               w_out, b_out,
               cam_v, id_v, dt_v,
               wc1_v, bc1_v, wi1_v, bi1_v, wc2_v, bc2_v, wi2_v, bi2_v,
               w_v, b_v):
        wid = lax.axis_index("s") * 2 + lax.axis_index("c")

        @pl.when(wid == 0)
        def _():
            pltpu.sync_copy(cam_h, cam_v)
            pltpu.sync_copy(id_h, id_v)
            pltpu.sync_copy(dt_h, dt_v)
            pltpu.sync_copy(wc1_h, wc1_v)
            pltpu.sync_copy(bc1_h, bc1_v)
            pltpu.sync_copy(wi1_h, wi1_v)
            pltpu.sync_copy(bi1_h, bi1_v)
            pltpu.sync_copy(wc2_h, wc2_v)
            pltpu.sync_copy(bc2_h, bc2_v)
            pltpu.sync_copy(wi2_h, wi2_v)
            pltpu.sync_copy(bi2_h, bi2_v)

            cam3 = cam_v[...] * 3
            id3 = id_v[...] * 3
            use1 = dt_v[...] == 0
            for c in range(3):
                w1 = (plsc.load_gather(wc1_v, [cam3 + c]) +
                      plsc.load_gather(wi1_v, [id3 + c]))
                w2 = (plsc.load_gather(wc2_v, [cam3 + c]) +
                      plsc.load_gather(wi2_v, [id3 + c]))
                b1 = (plsc.load_gather(bc1_v, [cam3 + c]) +
                      plsc.load_gather(bi1_v, [id3 + c]))
                b2 = (plsc.load_gather(bc2_v, [cam3 + c]) +
                      plsc.load_gather(bi2_v, [id3 + c]))
                w_v[c, :] = jnp.where(use1, w1, w2)
                b_v[c, :] = jnp.where(use1, b1, b2)
            pltpu.sync_copy(w_v, w_out)
            pltpu.sync_copy(b_v, b_out)

    return lookup(camindex, idindex, dataset_type,
                  wcam1f, bcam1f, wident1f, bident1f,
                  wcam2f, bcam2f, wident2f, bident2f)


NB = 4


def _affine_body(w_ref, b_ref, img_ref, out_ref):
    b_i = pl.program_id(0)
    for j in range(NB):
        for c in range(3):
            out_ref[j, c] = (img_ref[j, c] * w_ref[c, b_i * NB + j]
                             + b_ref[c, b_i * NB + j])


def _tc_affine(w, b, image):
    return pl.pallas_call(
        _affine_body,
        grid=(B // NB,),
        in_specs=[
            pl.BlockSpec(memory_space=pltpu.SMEM),
            pl.BlockSpec(memory_space=pltpu.SMEM),
            pl.BlockSpec((NB, 3, 512, 512), lambda bi: (bi, 0, 0, 0)),
        ],
        out_specs=pl.BlockSpec((NB, 3, 512, 512), lambda bi: (bi, 0, 0, 0)),
        out_shape=jax.ShapeDtypeStruct(image.shape, image.dtype),
        compiler_params=pltpu.CompilerParams(
            dimension_semantics=("parallel",)),
    )(w, b, image)


@jax.jit
def kernel(image, camindex, idindex, dataset_type,
           wcam1, bcam1, wident1, bident1,
           wcam2, bcam2, wident2, bident2):
    use_sc = False
    if use_sc:
        w, b = _sc_lookup(camindex, idindex, dataset_type,
                          wcam1.reshape(-1), bcam1.reshape(-1),
                          wident1.reshape(-1), bident1.reshape(-1),
                          wcam2.reshape(-1), bcam2.reshape(-1),
                          wident2.reshape(-1), bident2.reshape(-1))
    else:
        w1 = jnp.take(wcam1, camindex, axis=0) + jnp.take(wident1, idindex, axis=0)
        b1 = jnp.take(bcam1, camindex, axis=0) + jnp.take(bident1, idindex, axis=0)
        w2 = jnp.take(wcam2, camindex, axis=0) + jnp.take(wident2, idindex, axis=0)
        b2 = jnp.take(bcam2, camindex, axis=0) + jnp.take(bident2, idindex, axis=0)
        mask = (dataset_type == 0)[:, None]
        w = jnp.where(mask, w1, w2).T
        b = jnp.where(mask, b1, b2).T
    return _tc_affine(w, b, image)
